# Initial kernel scaffold; baseline (speedup 1.0000x reference)
import jax
import jax.numpy as jnp
from jax.experimental import pallas as pl


def kernel(x, edge_index, edge_weight, W1, b1, W2, b2, Wl, bl):
    raise NotImplementedError("write your pallas kernel here")


# speedup vs baseline: 1.3140x; 1.3140x over previous
import jax
import jax.numpy as jnp
from jax.experimental import pallas as pl

N = 10000
D = 128
H = 32


def _dense_block(x_ref, w_ref, b_ref, o_ref):
    o_ref[...] = x_ref[...] @ w_ref[...] + b_ref[...]


def _dense(x, W, b):
    n, d = x.shape
    h = W.shape[1]
    blk = 2000
    return pl.pallas_call(
        _dense_block,
        out_shape=jax.ShapeDtypeStruct((n, h), jnp.float32),
        grid=(n // blk,),
        in_specs=[
            pl.BlockSpec((blk, d), lambda i: (i, 0)),
            pl.BlockSpec((d, h), lambda i: (0, 0)),
            pl.BlockSpec((1, h), lambda i: (0, 0)),
        ],
        out_specs=pl.BlockSpec((blk, h), lambda i: (i, 0)),
    )(x, W, b.reshape(1, h))


def _gcn_conv(x, s, d, edge_weight, dinv, W, b):
    xw = _dense(x, W, jnp.zeros((W.shape[1],), jnp.float32))
    norm = dinv[s] * edge_weight * dinv[d]
    msg = xw[s] * norm[:, None]
    out = jax.ops.segment_sum(msg, d, num_segments=N)
    out = out + xw * (dinv * dinv)[:, None]
    return out + b


def kernel(x, edge_index, edge_weight, W1, b1, W2, b2, Wl, bl):
    s = edge_index[0]
    d = edge_index[1]
    deg = jax.ops.segment_sum(edge_weight, d, num_segments=N) + 1.0
    dinv = jax.lax.rsqrt(deg)
    h = jax.nn.relu(_gcn_conv(x, s, d, edge_weight, dinv, W1, b1))
    h = jax.nn.relu(_gcn_conv(h, s, d, edge_weight, dinv, W2, b2))
    out = _dense(h, Wl, bl).reshape(-1)
    return out


# speedup vs baseline: 24.5841x; 18.7097x over previous
import functools

import jax
import jax.numpy as jnp
from jax import lax
from jax.experimental import pallas as pl
from jax.experimental.pallas import tpu as pltpu
from jax.experimental.pallas import tpu_sc as plsc

N = 10000
E = 320000
D = 128
H = 32

NC = 2
NS = 16
NW = NC * NS
EPW = E // NW
CHUNK = 80
NCHUNK = EPW // CHUNK
N_PAD = 10240
RPS = N_PAD // NS


def _mesh():
    return plsc.VectorSubcoreMesh(core_axis_name="c", subcore_axis_name="s")


_SC_PARAMS = pltpu.CompilerParams(use_tc_tiling_on_sc=False)


def _sc_degree(d2, w2):
    @functools.partial(
        pl.kernel,
        out_type=jax.ShapeDtypeStruct((NC, N_PAD), jnp.float32),
        mesh=_mesh(),
        compiler_params=_SC_PARAMS,
        scratch_types=[
            pltpu.VMEM((NCHUNK, CHUNK), jnp.int32),
            pltpu.VMEM((NCHUNK, CHUNK), jnp.float32),
            pltpu.VMEM((RPS,), jnp.float32),
            pltpu.VMEM_SHARED((N_PAD,), jnp.float32),
        ],
    )
    def k(d_hbm, w_hbm, out_hbm, didx_v, w_v, zb_v, deg_sh):
        cid = lax.axis_index("c")
        sid = lax.axis_index("s")
        cbase = (cid * NS + sid) * NCHUNK

        @pl.loop(0, RPS, step=16)
        def _(i):
            zb_v[pl.ds(i, 16)] = jnp.zeros((16,), jnp.float32)

        pltpu.sync_copy(zb_v, deg_sh.at[pl.ds(sid * RPS, RPS)])
        pltpu.sync_copy(d_hbm.at[pl.ds(cbase, NCHUNK)], didx_v)
        pltpu.sync_copy(w_hbm.at[pl.ds(cbase, NCHUNK)], w_v)
        plsc.subcore_barrier()

        @pl.loop(0, NCHUNK)
        def _(c):
            pltpu.sync_copy(w_v.at[c], deg_sh.at[didx_v.at[c]], add=True)

        plsc.subcore_barrier()
        pltpu.sync_copy(deg_sh.at[pl.ds(sid * RPS, RPS)],
                        out_hbm.at[cid].at[pl.ds(sid * RPS, RPS)])

    return k(d2, w2)


def _sc_messages(u, s2, d2, w2):
    @functools.partial(
        pl.kernel,
        out_type=jax.ShapeDtypeStruct((NC, N_PAD, H), jnp.float32),
        mesh=_mesh(),
        compiler_params=_SC_PARAMS,
        scratch_types=[
            pltpu.VMEM((NCHUNK, CHUNK), jnp.int32),
            pltpu.VMEM((NCHUNK, CHUNK), jnp.int32),
            pltpu.VMEM((NCHUNK, CHUNK), jnp.float32),
            pltpu.VMEM((CHUNK, H), jnp.float32),
            pltpu.VMEM_SHARED((N_PAD, H), jnp.float32),
        ],
    )
    def k(u_hbm, s_hbm, d_hbm, w_hbm, out_hbm, sidx_v, didx_v, w_v, rows_v,
          acc_sh):
        cid = lax.axis_index("c")
        sid = lax.axis_index("s")
        cbase = (cid * NS + sid) * NCHUNK

        @pl.loop(0, CHUNK)
        def _(i):
            rows_v[i, pl.ds(0, 16)] = jnp.zeros((16,), jnp.float32)
            rows_v[i, pl.ds(16, 16)] = jnp.zeros((16,), jnp.float32)

        for z in range(RPS // CHUNK):
            pltpu.sync_copy(
                rows_v, acc_sh.at[pl.ds(sid * RPS + z * CHUNK, CHUNK)])

        pltpu.sync_copy(s_hbm.at[pl.ds(cbase, NCHUNK)], sidx_v)
        pltpu.sync_copy(d_hbm.at[pl.ds(cbase, NCHUNK)], didx_v)
        pltpu.sync_copy(w_hbm.at[pl.ds(cbase, NCHUNK)], w_v)
        plsc.subcore_barrier()

        @pl.loop(0, NCHUNK)
        def _(c):
            pltpu.sync_copy(u_hbm.at[sidx_v.at[c]], rows_v)
            for j in range(CHUNK // 16):
                w16 = w_v[c, pl.ds(j * 16, 16)]
                for i in range(16):
                    r = j * 16 + i
                    ws = w16[i]
                    rows_v[r, pl.ds(0, 16)] = rows_v[r, pl.ds(0, 16)] * ws
                    rows_v[r, pl.ds(16, 16)] = rows_v[r, pl.ds(16, 16)] * ws
            pltpu.sync_copy(rows_v, acc_sh.at[didx_v.at[c]], add=True)

        plsc.subcore_barrier()
        pltpu.sync_copy(acc_sh.at[pl.ds(sid * RPS, RPS)],
                        out_hbm.at[cid].at[pl.ds(sid * RPS, RPS)])

    return k(u, s2, d2, w2)


def _tc_layer1(x, W1, dinv):
    blk = 2000

    def body(x_ref, w_ref, dv_ref, xw_ref, u_ref):
        xw = jnp.dot(x_ref[...], w_ref[...],
                     preferred_element_type=jnp.float32)
        xw_ref[...] = xw
        u_ref[...] = xw * dv_ref[...]

    return pl.pallas_call(
        body,
        out_shape=(jax.ShapeDtypeStruct((N, H), jnp.float32),
                   jax.ShapeDtypeStruct((N, H), jnp.float32)),
        grid=(N // blk,),
        in_specs=[
            pl.BlockSpec((blk, D), lambda i: (i, 0)),
            pl.BlockSpec((D, H), lambda i: (0, 0)),
            pl.BlockSpec((blk, 1), lambda i: (i, 0)),
        ],
        out_specs=(pl.BlockSpec((blk, H), lambda i: (i, 0)),
                   pl.BlockSpec((blk, H), lambda i: (i, 0))),
    )(x, W1, dinv)


def _tc_layer2(accs, xw1, dinv, b1, W2):
    blk = 2000

    def body(a_ref, xw_ref, dv_ref, b_ref, w2_ref, xw2_ref, u2_ref):
        dv = dv_ref[...]
        h = jnp.maximum(
            (a_ref[0] + a_ref[1]) * dv + xw_ref[...] * (dv * dv) + b_ref[...],
            0.0)
        xw2 = jnp.dot(h, w2_ref[...], preferred_element_type=jnp.float32)
        xw2_ref[...] = xw2
        u2_ref[...] = xw2 * dv

    return pl.pallas_call(
        body,
        out_shape=(jax.ShapeDtypeStruct((N, H), jnp.float32),
                   jax.ShapeDtypeStruct((N, H), jnp.float32)),
        grid=(N // blk,),
        in_specs=[
            pl.BlockSpec((NC, blk, H), lambda i: (0, i, 0)),
            pl.BlockSpec((blk, H), lambda i: (i, 0)),
            pl.BlockSpec((blk, 1), lambda i: (i, 0)),
            pl.BlockSpec((1, H), lambda i: (0, 0)),
            pl.BlockSpec((H, H), lambda i: (0, 0)),
        ],
        out_specs=(pl.BlockSpec((blk, H), lambda i: (i, 0)),
                   pl.BlockSpec((blk, H), lambda i: (i, 0))),
    )(accs, xw1, dinv, b1, W2)


def _tc_final(accs, xw2, dinv, b2, Wl, bl):
    blk = 2000

    def body(a_ref, xw_ref, dv_ref, b_ref, wl_ref, bl_ref, o_ref):
        dv = dv_ref[...]
        h = jnp.maximum(
            (a_ref[0] + a_ref[1]) * dv + xw_ref[...] * (dv * dv) + b_ref[...],
            0.0)
        o_ref[...] = jnp.dot(h, wl_ref[...],
                             preferred_element_type=jnp.float32) + bl_ref[...]

    return pl.pallas_call(
        body,
        out_shape=jax.ShapeDtypeStruct((N, 1), jnp.float32),
        grid=(N // blk,),
        in_specs=[
            pl.BlockSpec((NC, blk, H), lambda i: (0, i, 0)),
            pl.BlockSpec((blk, H), lambda i: (i, 0)),
            pl.BlockSpec((blk, 1), lambda i: (i, 0)),
            pl.BlockSpec((1, H), lambda i: (0, 0)),
            pl.BlockSpec((H, 1), lambda i: (0, 0)),
            pl.BlockSpec((1, 1), lambda i: (0, 0)),
        ],
        out_specs=pl.BlockSpec((blk, 1), lambda i: (i, 0)),
    )(accs, xw2, dinv, b2, Wl, bl)


def kernel(x, edge_index, edge_weight, W1, b1, W2, b2, Wl, bl):
    s2 = edge_index[0].reshape(E // CHUNK, CHUNK)
    d2 = edge_index[1].reshape(E // CHUNK, CHUNK)
    w2 = edge_weight.reshape(E // CHUNK, CHUNK)

    degp = _sc_degree(d2, w2)
    deg = degp[0, :N] + degp[1, :N] + 1.0
    dinv = lax.rsqrt(deg).reshape(N, 1)

    xw1, u1 = _tc_layer1(x, W1, dinv)
    acc1 = _sc_messages(u1, s2, d2, w2)[:, :N, :]
    xw2, u2 = _tc_layer2(acc1, xw1, dinv, b1.reshape(1, H), W2)
    acc2 = _sc_messages(u2, s2, d2, w2)[:, :N, :]
    out = _tc_final(acc2, xw2, dinv, b2.reshape(1, H), Wl, bl.reshape(1, 1))
    return out.reshape(-1)
